# baseline (speedup 1.0000x reference)
import jax
import jax.numpy as jnp
from jax import lax
from jax.experimental import pallas as pl
from jax.experimental.pallas import tpu as pltpu
from jax.experimental.pallas import tpu_sc as plsc

IN_FEATURES = 1000000
OUT_FEATURES = 64
BATCH = 16384
NUM_FIELDS = 26

_INFO = plsc.get_sparse_core_info()
NC = _INFO.num_cores
NS = _INFO.num_subcores
LANES = 16
CPSC = OUT_FEATURES // NC
BPT = BATCH // NS
NV = BPT // LANES
CHUNK = 62464
LCHUNK = 62976
TAIL_OFF = 15 * CHUNK + LCHUNK
TAIL = IN_FEATURES - TAIL_OFF


SPLITS = ((0, 7), (7, 14), (14, 20), (20, 26))


def _body(wt_hbm, idx_hbm, brep_hbm, out_hbm, row_sh, idxs0, idxs1, idxs2,
          idxs3, gath0, gath1, gath2, gath3, outrow_v, brep_v, tail_v,
          lsem, gsem0, gsem1, gsem2, gsem3, osem, tsem):
    idxs = (idxs0, idxs1, idxs2, idxs3)
    gaths = (gath0, gath1, gath2, gath3)
    gsems = (gsem0, gsem1, gsem2, gsem3)
    sc = lax.axis_index("c")
    tile = lax.axis_index("s")
    b0 = pl.multiple_of(tile * BPT, BPT)
    off = pl.multiple_of(tile * CHUNK, 1024)

    def fire_load(cc):
        c = sc * CPSC + cc

        @pl.when(tile != NS - 1)
        def _():
            pltpu.async_copy(
                wt_hbm.at[c, pl.ds(off, CHUNK)],
                row_sh.at[pl.ds(off, CHUNK)],
                lsem,
            )

        @pl.when(tile == NS - 1)
        def _():
            loff = pl.multiple_of((NS - 1) * CHUNK, 1024)
            pltpu.async_copy(
                wt_hbm.at[c, pl.ds(loff, LCHUNK)],
                row_sh.at[pl.ds(loff, LCHUNK)],
                lsem,
            )

        @pl.when(tile == 0)
        def _():
            c8 = pl.multiple_of((c // 8) * 8, 8)
            pltpu.async_copy(
                wt_hbm.at[pl.ds(c8, 8), pl.ds(TAIL_OFF, TAIL)], tail_v, tsem
            ).wait()
            pltpu.async_copy(
                tail_v.at[lax.rem(c, 8)],
                row_sh.at[pl.ds(TAIL_OFF, TAIL)],
                tsem,
            ).wait()

    def drain_load():
        @pl.when(tile != NS - 1)
        def _():
            pltpu.make_async_copy(
                wt_hbm.at[0, pl.ds(0, CHUNK)],
                row_sh.at[pl.ds(0, CHUNK)],
                lsem,
            ).wait()

        @pl.when(tile == NS - 1)
        def _():
            pltpu.make_async_copy(
                wt_hbm.at[0, pl.ds(0, LCHUNK)],
                row_sh.at[pl.ds(0, LCHUNK)],
                lsem,
            ).wait()

    fire_load(0)
    pltpu.sync_copy(brep_hbm, brep_v)
    stage = []
    for k, (lo, hi) in enumerate(SPLITS):
        for j in range(lo, hi):
            stage.append(pltpu.async_copy(
                idx_hbm.at[j, pl.ds(b0, BPT)],
                idxs[k].at[pl.ds((j - lo) * BPT, BPT)],
                gsems[k],
            ))
    for cp in stage:
        cp.wait()
    drain_load()
    plsc.subcore_barrier()

    for cc in range(CPSC):
        c = sc * CPSC + cc
        oslot = cc % 2

        gcps = [
            pltpu.async_copy(row_sh.at[idxs[k]], gaths[k], gsems[k])
            for k in range(4)
        ]
        for cp in gcps:
            cp.wait()
        plsc.subcore_barrier()
        if cc + 1 < CPSC:
            fire_load(cc + 1)

        if cc >= 2:
            pltpu.make_async_copy(
                wt_hbm.at[0, pl.ds(0, BPT)], outrow_v.at[oslot], osem
            ).wait()

        bias_vec = brep_v[pl.ds(c * LANES, LANES)]

        @plsc.parallel_loop(0, NV)
        def _reduce(v):
            bo = v * LANES
            acc = bias_vec
            for k, (lo, hi) in enumerate(SPLITS):
                for j in range(lo, hi):
                    acc = acc + gaths[k][pl.ds((j - lo) * BPT + bo, LANES)]
            outrow_v[oslot, pl.ds(bo, LANES)] = acc

        pltpu.async_copy(
            outrow_v.at[oslot], out_hbm.at[c, pl.ds(b0, BPT)], osem
        )

        if cc + 1 < CPSC:
            drain_load()
            plsc.subcore_barrier()

    for oslot in range(2):
        pltpu.make_async_copy(
            wt_hbm.at[0, pl.ds(0, BPT)], outrow_v.at[oslot], osem
        ).wait()


@jax.jit
def _run(wt, idx_t, bias_rep):
    kern = pl.kernel(
        _body,
        mesh=plsc.VectorSubcoreMesh(core_axis_name="c", subcore_axis_name="s"),
        compiler_params=pltpu.CompilerParams(use_tc_tiling_on_sc=True),
        out_type=jax.ShapeDtypeStruct((OUT_FEATURES, BATCH), jnp.float32),
        scratch_types=[
            pltpu.VMEM_SHARED((IN_FEATURES,), jnp.float32),
            pltpu.VMEM((7 * BPT,), jnp.int32),
            pltpu.VMEM((7 * BPT,), jnp.int32),
            pltpu.VMEM((6 * BPT,), jnp.int32),
            pltpu.VMEM((6 * BPT,), jnp.int32),
            pltpu.VMEM((7 * BPT,), jnp.float32),
            pltpu.VMEM((7 * BPT,), jnp.float32),
            pltpu.VMEM((6 * BPT,), jnp.float32),
            pltpu.VMEM((6 * BPT,), jnp.float32),
            pltpu.VMEM((2, BPT), jnp.float32),
            pltpu.VMEM((OUT_FEATURES * LANES,), jnp.float32),
            pltpu.VMEM((8, TAIL), jnp.float32),
            pltpu.SemaphoreType.DMA,
            pltpu.SemaphoreType.DMA,
            pltpu.SemaphoreType.DMA,
            pltpu.SemaphoreType.DMA,
            pltpu.SemaphoreType.DMA,
            pltpu.SemaphoreType.DMA,
            pltpu.SemaphoreType.DMA,
        ],
    )
    return kern(wt, idx_t, bias_rep)


def kernel(indices, weight, bias):
    wt = weight.T
    idx_t = jnp.asarray(indices, dtype=jnp.int32).T
    idx_t = jnp.pad(idx_t, ((0, 32 - NUM_FIELDS), (0, 0)))
    bias_rep = jnp.repeat(bias, LANES)
    out_t = _run(wt, idx_t, bias_rep)
    return out_t.T
